# baseline (speedup 1.0000x reference)
import functools

import jax
import jax.numpy as jnp
from jax import lax
from jax.experimental import pallas as pl
from jax.experimental.pallas import tpu as pltpu
from jax.experimental.pallas import tpu_sc as plsc

B = 2
C_IN = 10
P = 12000
NPTS = 100
C_OUT = 64
GRID = 500
CELLS = GRID * GRID
P_PAD = P + 8
C_PAD = 128
PBLK = 120
CHUNK = 400
TOTAL_ROWS = B * CELLS
N_CHUNKS = TOTAL_ROWS // CHUNK
TBLK = 2048


def _encode_body(f_ref, w_ref, scale_ref, shift_ref, o_ref):
    f = f_ref[0].reshape(C_IN, PBLK * NPTS)
    w = w_ref[...]
    y = lax.dot_general(f, w, (((0,), (1,)), ((), ())),
                        preferred_element_type=jnp.float32)
    y = jnp.maximum(y * scale_ref[...] + shift_ref[...], 0.0)
    y = y.reshape(PBLK, NPTS, C_PAD)
    o_ref[0] = jnp.max(y, axis=1)


def _encode(pillar_feats, w, scale, shift):
    return pl.pallas_call(
        _encode_body,
        grid=(B, P // PBLK),
        in_specs=[
            pl.BlockSpec((1, C_IN, PBLK, NPTS), lambda b, i: (b, 0, i, 0)),
            pl.BlockSpec((C_PAD, C_IN), lambda b, i: (0, 0)),
            pl.BlockSpec((1, C_PAD), lambda b, i: (0, 0)),
            pl.BlockSpec((1, C_PAD), lambda b, i: (0, 0)),
        ],
        out_specs=pl.BlockSpec((1, PBLK, C_PAD), lambda b, i: (b, i, 0)),
        out_shape=jax.ShapeDtypeStruct((B, P, C_PAD), jnp.float32),
    )(pillar_feats, w, scale, shift)


def _make_sc_gather():
    info = plsc.get_sparse_core_info()
    nc, ns = info.num_cores, info.num_subcores
    nw = nc * ns
    mesh = plsc.VectorSubcoreMesh(core_axis_name="c", subcore_axis_name="s")
    n_rounds = -(-N_CHUNKS // nw)

    @functools.partial(
        pl.kernel,
        mesh=mesh,
        out_type=jax.ShapeDtypeStruct((TOTAL_ROWS, C_PAD), jnp.float32),
        scratch_types=[
            pltpu.VMEM((CHUNK,), jnp.int32),
            pltpu.VMEM((CHUNK, C_PAD), jnp.float32),
            pltpu.SemaphoreType.DMA,
        ],
    )
    def sc_gather(enc_hbm, gidx_hbm, out_hbm, idx_v, rows_v, sem):
        wid = lax.axis_index("s") * nc + lax.axis_index("c")
        for j in range(n_rounds):
            k = j * nw + wid

            @pl.when(k < N_CHUNKS)
            def _():
                base = k * CHUNK
                pltpu.sync_copy(gidx_hbm.at[pl.ds(base, CHUNK)], idx_v)
                pltpu.async_copy(enc_hbm.at[idx_v], rows_v, sem).wait()
                pltpu.sync_copy(rows_v, out_hbm.at[pl.ds(base, CHUNK)])

    return sc_gather


def _transpose_body(i_ref, o_ref):
    o_ref[0] = i_ref[0][:, :C_OUT].T


def _to_channel_major(bev_rows):
    return pl.pallas_call(
        _transpose_body,
        grid=(B, -(-CELLS // TBLK)),
        in_specs=[
            pl.BlockSpec((1, TBLK, C_PAD), lambda b, i: (b, i, 0)),
        ],
        out_specs=pl.BlockSpec((1, C_OUT, TBLK), lambda b, i: (b, 0, i)),
        out_shape=jax.ShapeDtypeStruct((B, C_OUT, CELLS), jnp.float32),
    )(bev_rows)


def kernel(pillar_feats, W, gamma, beta, running_mean, running_var, pillar_idxs):
    eps = 1e-5
    scale = gamma / jnp.sqrt(running_var + eps)
    shift = beta - running_mean * scale
    w_pad = jnp.pad(W, ((0, C_PAD - C_OUT), (0, 0)))
    scale = jnp.pad(scale, (0, C_PAD - C_OUT)).reshape(1, C_PAD)
    shift = jnp.pad(shift, (0, C_PAD - C_OUT)).reshape(1, C_PAD)

    enc = _encode(pillar_feats, w_pad, scale, shift)
    enc_flat = jnp.pad(enc, ((0, 0), (1, 7), (0, 0))).reshape(B * P_PAD, C_PAD)

    flat_idx = pillar_idxs[..., 0] * GRID + pillar_idxs[..., 1]
    pids = jnp.arange(1, P + 1, dtype=jnp.int32)
    win = jax.vmap(
        lambda fi: jnp.zeros((CELLS,), jnp.int32).at[fi].max(pids)
    )(flat_idx)
    gidx = win + (jnp.arange(B, dtype=jnp.int32) * P_PAD)[:, None]
    gidx = gidx.reshape(-1)

    bev_rows = _make_sc_gather()(enc_flat, gidx)
    bev = _to_channel_major(bev_rows.reshape(B, CELLS, C_PAD))

    return bev.reshape(B, C_OUT, GRID, GRID)
